# baseline (speedup 1.0000x reference)
import functools

import jax
import jax.numpy as jnp
from jax import lax
from jax.experimental import pallas as pl
from jax.experimental.pallas import tpu as pltpu
from jax.experimental.pallas import tpu_sc as plsc

NC = 2
NS = 16
NW = NC * NS

B = 16384 * 26
D = 64

RPG = 128
GPC = 4
CH = RPG * GPC
ROWS_PER_W = B // NW
CHUNKS = ROWS_PER_W // CH
IDX_ROWS_PER_W = ROWS_PER_W // RPG


def kernel(input, embedding_weight):
    idx2d = input.reshape(B // RPG, RPG)

    mesh = plsc.VectorSubcoreMesh(core_axis_name="c", subcore_axis_name="s")

    @functools.partial(
        pl.kernel,
        out_type=jax.ShapeDtypeStruct((B, D // 2), jnp.int32),
        mesh=mesh,
        scratch_types=[
            pltpu.VMEM((GPC, RPG), jnp.int32),
            pltpu.VMEM((CH, D), jnp.float32),
            pltpu.VMEM((CH, D // 2), jnp.int32),
            pltpu.SemaphoreType.DMA,
        ],
        compiler_params=pltpu.CompilerParams(
            needs_layout_passes=False, use_tc_tiling_on_sc=False
        ),
    )
    def emb(idx_hbm, table_hbm, out_hbm, idx_v, rows_v, out_v, sem):
        wid = lax.axis_index("s") * NC + lax.axis_index("c")
        row0 = wid * ROWS_PER_W
        irow0 = wid * IDX_ROWS_PER_W

        @pl.loop(0, CHUNKS)
        def chunk_body(t):
            pltpu.sync_copy(idx_hbm.at[pl.ds(irow0 + t * GPC, GPC)], idx_v)
            copies = []
            for g in range(GPC):
                copies.append(
                    pltpu.async_copy(
                        table_hbm.at[idx_v.at[g]],
                        rows_v.at[pl.ds(g * RPG, RPG)],
                        sem,
                    )
                )
            for cp in copies:
                cp.wait()

            @pl.loop(0, CH)
            def conv(j):
                row_idx = jnp.full((16,), j, dtype=jnp.int32)
                lane2 = lax.iota(jnp.int32, 16) * 2
                for h in range(2):
                    evens = plsc.load_gather(rows_v, [row_idx, lane2 + 32 * h])
                    odds = plsc.load_gather(
                        rows_v, [row_idx, lane2 + (32 * h + 1)]
                    )
                    p = plsc.pack(
                        evens, odds, format=plsc.PackFormat.INTERLEAVED
                    )
                    out_v[j, pl.ds(16 * h, 16)] = plsc.bitcast(p, jnp.int32)

            pltpu.sync_copy(out_v, out_hbm.at[pl.ds(row0 + t * CH, CH)])

    packed = emb(idx2d, embedding_weight)
    out_bf = lax.bitcast_convert_type(packed, jnp.bfloat16)
    return out_bf.reshape(16384, 26, D)


# speedup vs baseline: 1.0363x; 1.0363x over previous
import functools

import jax
import jax.numpy as jnp
from jax import lax
from jax.experimental import pallas as pl
from jax.experimental.pallas import tpu as pltpu
from jax.experimental.pallas import tpu_sc as plsc

NC = 2
NS = 16
NW = NC * NS

B = 16384 * 26
D = 64

RPG = 128
GPC = 4
CH = RPG * GPC
ROWS_PER_W = B // NW
CHUNKS = ROWS_PER_W // CH
IDX_ROWS_PER_W = ROWS_PER_W // RPG


def kernel(input, embedding_weight):
    idx2d = input.reshape(B // RPG, RPG)

    mesh = plsc.VectorSubcoreMesh(core_axis_name="c", subcore_axis_name="s")

    @functools.partial(
        pl.kernel,
        out_type=jax.ShapeDtypeStruct((B, D // 2), jnp.int32),
        mesh=mesh,
        scratch_types=[
            pltpu.VMEM((IDX_ROWS_PER_W, RPG), jnp.int32),
            pltpu.VMEM((2, CH, D), jnp.float32),
            pltpu.VMEM((2, CH, D // 2), jnp.int32),
            pltpu.SemaphoreType.DMA,
            pltpu.SemaphoreType.DMA,
            pltpu.SemaphoreType.DMA,
            pltpu.SemaphoreType.DMA,
        ],
        compiler_params=pltpu.CompilerParams(
            needs_layout_passes=False, use_tc_tiling_on_sc=False
        ),
    )
    def emb(idx_hbm, table_hbm, out_hbm, idx_v, rows_v, out_v,
            sg0, sg1, so0, so1):
        wid = lax.axis_index("s") * NC + lax.axis_index("c")
        row0 = wid * ROWS_PER_W
        irow0 = wid * IDX_ROWS_PER_W
        sgs = (sg0, sg1)
        sos = (so0, so1)

        pltpu.sync_copy(idx_hbm.at[pl.ds(irow0, IDX_ROWS_PER_W)], idx_v)

        def fire_gathers(t, b):
            for g in range(GPC):
                pltpu.async_copy(
                    table_hbm.at[idx_v.at[t * GPC + g]],
                    rows_v.at[b, pl.ds(g * RPG, RPG)],
                    sgs[b],
                )

        def wait_gathers(b):
            pltpu.make_async_copy(
                table_hbm.at[pl.ds(0, CH)], rows_v.at[b], sgs[b]
            ).wait()

        def fire_out(t, b):
            pltpu.async_copy(
                out_v.at[b], out_hbm.at[pl.ds(row0 + t * CH, CH)], sos[b]
            )

        def wait_out(b):
            pltpu.make_async_copy(
                out_hbm.at[pl.ds(0, CH)], out_v.at[b], sos[b]
            ).wait()

        def convert(b):
            @pl.loop(0, CH, unroll=2)
            def conv(j):
                row_idx = jnp.full((16,), j, dtype=jnp.int32)
                lane2 = lax.iota(jnp.int32, 16) * 2
                for h in range(2):
                    evens = plsc.load_gather(
                        rows_v.at[b], [row_idx, lane2 + 32 * h]
                    )
                    odds = plsc.load_gather(
                        rows_v.at[b], [row_idx, lane2 + (32 * h + 1)]
                    )
                    p = plsc.pack(
                        evens, odds, format=plsc.PackFormat.INTERLEAVED
                    )
                    out_v[b, j, pl.ds(16 * h, 16)] = plsc.bitcast(p, jnp.int32)

        fire_gathers(0, 0)

        @pl.loop(0, CHUNKS, step=2)
        def pair(t0):
            for b in range(2):
                t = t0 + b

                @pl.when(t + 1 < CHUNKS)
                def _():
                    fire_gathers(t + 1, b ^ 1)

                wait_gathers(b)

                @pl.when(t >= 2)
                def _():
                    wait_out(b)

                convert(b)
                fire_out(t, b)

        wait_out(0)
        wait_out(1)

    packed = emb(idx2d, embedding_weight)
    out_bf = lax.bitcast_convert_type(packed, jnp.bfloat16)
    return out_bf.reshape(16384, 26, D)


# speedup vs baseline: 2.1236x; 2.0491x over previous
import functools

import jax
import jax.numpy as jnp
from jax import lax
from jax.experimental import pallas as pl
from jax.experimental.pallas import tpu as pltpu
from jax.experimental.pallas import tpu_sc as plsc

NC = 2
NS = 16
NW = NC * NS

B = 16384 * 26
D = 64

RPG = 128
GPC = 4
CH = RPG * GPC
ROWS_PER_W = B // NW
CHUNKS = ROWS_PER_W // CH
IDX_ROWS_PER_W = ROWS_PER_W // RPG


def kernel(input, embedding_weight):
    idx2d = input.reshape(B // RPG, RPG)

    mesh = plsc.VectorSubcoreMesh(core_axis_name="c", subcore_axis_name="s")

    @functools.partial(
        pl.kernel,
        out_type=jax.ShapeDtypeStruct((B, D), jnp.bfloat16),
        mesh=mesh,
        scratch_types=[
            pltpu.VMEM((IDX_ROWS_PER_W, RPG), jnp.int32),
            pltpu.VMEM((2, CH, D), jnp.float32),
            pltpu.VMEM((2, CH, D), jnp.bfloat16),
            pltpu.SemaphoreType.DMA,
            pltpu.SemaphoreType.DMA,
            pltpu.SemaphoreType.DMA,
            pltpu.SemaphoreType.DMA,
        ],
        compiler_params=pltpu.CompilerParams(
            needs_layout_passes=False, use_tc_tiling_on_sc=False
        ),
    )
    def emb(idx_hbm, table_hbm, out_hbm, idx_v, rows_v, out_v,
            sg0, sg1, so0, so1):
        wid = lax.axis_index("s") * NC + lax.axis_index("c")
        row0 = wid * ROWS_PER_W
        irow0 = wid * IDX_ROWS_PER_W
        sgs = (sg0, sg1)
        sos = (so0, so1)

        pltpu.sync_copy(idx_hbm.at[pl.ds(irow0, IDX_ROWS_PER_W)], idx_v)

        def fire_gathers(t, b):
            for g in range(GPC):
                pltpu.async_copy(
                    table_hbm.at[idx_v.at[t * GPC + g]],
                    rows_v.at[b, pl.ds(g * RPG, RPG)],
                    sgs[b],
                )

        def wait_gathers(b):
            pltpu.make_async_copy(
                table_hbm.at[pl.ds(0, CH)], rows_v.at[b], sgs[b]
            ).wait()

        def fire_out(t, b):
            pltpu.async_copy(
                out_v.at[b], out_hbm.at[pl.ds(row0 + t * CH, CH)], sos[b]
            )

        def wait_out(b):
            pltpu.make_async_copy(
                out_hbm.at[pl.ds(0, CH)], out_v.at[b], sos[b]
            ).wait()

        def convert(b):
            @pl.loop(0, CH, unroll=2)
            def conv(j):
                row_idx = jnp.full((16,), j, dtype=jnp.int32)
                lane2 = lax.iota(jnp.int32, 16) * 2
                for h in range(2):
                    evens = plsc.load_gather(
                        rows_v.at[b], [row_idx, lane2 + 32 * h]
                    )
                    odds = plsc.load_gather(
                        rows_v.at[b], [row_idx, lane2 + (32 * h + 1)]
                    )
                    out_v[b, j, pl.ds(32 * h, 32)] = plsc.pack(
                        evens, odds, format=plsc.PackFormat.INTERLEAVED
                    )

        fire_gathers(0, 0)

        @pl.loop(0, CHUNKS, step=2)
        def pair(t0):
            for b in range(2):
                t = t0 + b

                @pl.when(t + 1 < CHUNKS)
                def _():
                    fire_gathers(t + 1, b ^ 1)

                wait_gathers(b)

                @pl.when(t >= 2)
                def _():
                    wait_out(b)

                convert(b)
                fire_out(t, b)

        wait_out(0)
        wait_out(1)

    out = emb(idx2d, embedding_weight)
    return out.reshape(16384, 26, D)


# speedup vs baseline: 2.1305x; 1.0032x over previous
import functools

import jax
import jax.numpy as jnp
from jax import lax
from jax.experimental import pallas as pl
from jax.experimental.pallas import tpu as pltpu
from jax.experimental.pallas import tpu_sc as plsc

NC = 2
NS = 16
NW = NC * NS

B = 16384 * 26
D = 64

RPG = 128
GPC = 4
CH = RPG * GPC
ROWS_PER_W = B // NW
CHUNKS = ROWS_PER_W // CH
IDX_ROWS_PER_W = ROWS_PER_W // RPG


def kernel(input, embedding_weight):
    idx2d = input.reshape(B // RPG, RPG)

    mesh = plsc.VectorSubcoreMesh(core_axis_name="c", subcore_axis_name="s")

    @functools.partial(
        pl.kernel,
        out_type=jax.ShapeDtypeStruct((B, D), jnp.bfloat16),
        mesh=mesh,
        scratch_types=[
            pltpu.VMEM((IDX_ROWS_PER_W, RPG), jnp.int32),
            pltpu.VMEM((2, CH, D), jnp.float32),
            pltpu.VMEM((2, CH, D), jnp.bfloat16),
            pltpu.SemaphoreType.DMA,
            pltpu.SemaphoreType.DMA,
            pltpu.SemaphoreType.DMA,
            pltpu.SemaphoreType.DMA,
        ],
        compiler_params=pltpu.CompilerParams(
            needs_layout_passes=False,
            use_tc_tiling_on_sc=False,
            skip_device_barrier=True,
        ),
    )
    def emb(idx_hbm, table_hbm, out_hbm, idx_v, rows_v, out_v,
            sg0, sg1, so0, so1):
        wid = lax.axis_index("s") * NC + lax.axis_index("c")
        row0 = wid * ROWS_PER_W
        irow0 = wid * IDX_ROWS_PER_W
        sgs = (sg0, sg1)
        sos = (so0, so1)

        pltpu.sync_copy(idx_hbm.at[pl.ds(irow0, IDX_ROWS_PER_W)], idx_v)

        def fire_gathers(t, b):
            for g in range(GPC):
                pltpu.async_copy(
                    table_hbm.at[idx_v.at[t * GPC + g]],
                    rows_v.at[b, pl.ds(g * RPG, RPG)],
                    sgs[b],
                )

        def wait_gathers(b):
            pltpu.make_async_copy(
                table_hbm.at[pl.ds(0, CH)], rows_v.at[b], sgs[b]
            ).wait()

        def fire_out(t, b):
            pltpu.async_copy(
                out_v.at[b], out_hbm.at[pl.ds(row0 + t * CH, CH)], sos[b]
            )

        def wait_out(b):
            pltpu.make_async_copy(
                out_hbm.at[pl.ds(0, CH)], out_v.at[b], sos[b]
            ).wait()

        lane2 = lax.iota(jnp.int32, 16) * 2
        offs = [lane2, lane2 + 1, lane2 + 32, lane2 + 33]

        def convert(b):
            @pl.loop(0, CH, unroll=4)
            def conv(j):
                row = jnp.full((16,), j, dtype=jnp.int32)
                for h in range(2):
                    evens = plsc.load_gather(rows_v.at[b], [row, offs[2 * h]])
                    odds = plsc.load_gather(
                        rows_v.at[b], [row, offs[2 * h + 1]]
                    )
                    out_v[b, j, pl.ds(32 * h, 32)] = plsc.pack(
                        evens, odds, format=plsc.PackFormat.INTERLEAVED
                    )

        fire_gathers(0, 0)

        @pl.loop(0, CHUNKS, step=2)
        def pair(t0):
            for b in range(2):
                t = t0 + b

                @pl.when(t + 1 < CHUNKS)
                def _():
                    fire_gathers(t + 1, b ^ 1)

                wait_gathers(b)

                @pl.when(t >= 2)
                def _():
                    wait_out(b)

                convert(b)
                fire_out(t, b)

        wait_out(0)
        wait_out(1)

    out = emb(idx2d, embedding_weight)
    return out.reshape(16384, 26, D)
